# baseline (speedup 1.0000x reference)
import functools
import math

import jax
import jax.numpy as jnp
import numpy as np
from jax.experimental import pallas as pl
from jax.experimental.pallas import tpu as pltpu

B, S, D = 1, 2048, 1024
H = 16
HD = D // H
E = 8
DFF = 4 * D
CAP = math.ceil(1.25 * S / E)
MEM_LEN = 256
CMEM_LEN = 128
KF = 8
DFB = DFF // KF


def _ln(x, g, b):
    m = jnp.mean(x, axis=-1, keepdims=True)
    v = jnp.mean((x - m) ** 2, axis=-1, keepdims=True)
    return (x - m) / jnp.sqrt(v + 1e-5) * g + b


def _mm(a, w, bias=None, *, act=None, res=None, bm=256, bn=512):
    M, K = a.shape
    N = w.shape[1]
    bm = min(bm, M)
    bn = min(bn, N)
    operands = [a, w]
    specs = [
        pl.BlockSpec((bm, K), lambda i, j: (i, 0)),
        pl.BlockSpec((K, bn), lambda i, j: (0, j)),
    ]
    has_bias = bias is not None
    has_res = res is not None
    if has_bias:
        operands.append(bias.reshape(1, N))
        specs.append(pl.BlockSpec((1, bn), lambda i, j: (0, j)))
    if has_res:
        operands.append(res)
        specs.append(pl.BlockSpec((bm, bn), lambda i, j: (i, j)))

    def kfn(*refs):
        it = iter(refs)
        a_ref = next(it)
        w_ref = next(it)
        b_ref = next(it) if has_bias else None
        r_ref = next(it) if has_res else None
        o_ref = next(it)
        o = jnp.dot(a_ref[...], w_ref[...], preferred_element_type=jnp.float32)
        if has_bias:
            o = o + b_ref[...]
        if act == "relu":
            o = jnp.maximum(o, 0.0)
        if has_res:
            o = o + r_ref[...]
        o_ref[...] = o

    return pl.pallas_call(
        kfn,
        grid=(M // bm, N // bn),
        in_specs=specs,
        out_specs=pl.BlockSpec((bm, bn), lambda i, j: (i, j)),
        out_shape=jax.ShapeDtypeStruct((M, N), jnp.float32),
    )(*operands)


def _attn_scores_kernel(q_ref, k_ref, o_ref, *, bq):
    i = pl.program_id(1)
    s = jax.lax.dot_general(
        q_ref[0], k_ref[0], (((1,), (1,)), ((), ())),
        preferred_element_type=jnp.float32,
    ) / math.sqrt(HD)
    row = i * bq + jax.lax.broadcasted_iota(jnp.int32, (bq, S), 0)
    col = jax.lax.broadcasted_iota(jnp.int32, (bq, S), 1)
    o_ref[0] = jnp.where(col > row, -1e30, s)


def _attn_scores(q, k, bq=512):
    return pl.pallas_call(
        functools.partial(_attn_scores_kernel, bq=bq),
        grid=(H, S // bq),
        in_specs=[
            pl.BlockSpec((1, bq, HD), lambda h, i: (h, i, 0)),
            pl.BlockSpec((1, S, HD), lambda h, i: (h, 0, 0)),
        ],
        out_specs=pl.BlockSpec((1, bq, S), lambda h, i: (h, i, 0)),
        out_shape=jax.ShapeDtypeStruct((H, S, S), jnp.float32),
    )(q, k)


def _attn_av_kernel(w_ref, v_ref, o_ref):
    o_ref[0] = jnp.dot(w_ref[0], v_ref[0], preferred_element_type=jnp.float32)


def _attn_av(w, v, bq=512):
    return pl.pallas_call(
        _attn_av_kernel,
        grid=(H, S // bq),
        in_specs=[
            pl.BlockSpec((1, bq, S), lambda h, i: (h, i, 0)),
            pl.BlockSpec((1, S, HD), lambda h, i: (h, 0, 0)),
        ],
        out_specs=pl.BlockSpec((1, bq, HD), lambda h, i: (h, i, 0)),
        out_shape=jax.ShapeDtypeStruct((H, S, HD), jnp.float32),
    )(w, v)


def _expert_kernel(h3_ref, ti_ref, ts_ref, ew1_ref, eb1_ref,
                   ew2_ref, eb2_ref, o_ref, disp_ref, acc_ref):
    e = pl.program_id(0)
    kf = pl.program_id(1)

    @pl.when(jnp.logical_and(e == 0, kf == 0))
    def _():
        o_ref[...] = jnp.zeros_like(o_ref)

    rows = jax.lax.broadcasted_iota(jnp.int32, (S, CAP), 0)
    oh = (rows == ti_ref[0]).astype(jnp.float32)

    @pl.when(kf == 0)
    def _():
        disp_ref[...] = jax.lax.dot_general(
            oh, h3_ref[...], (((0,), (0,)), ((), ())),
            precision=jax.lax.Precision.HIGHEST,
            preferred_element_type=jnp.float32,
        )

    h1 = jnp.maximum(
        jnp.dot(disp_ref[...], ew1_ref[0], preferred_element_type=jnp.float32)
        + eb1_ref[0, 0], 0.0)
    contrib = jnp.dot(h1, ew2_ref[0], preferred_element_type=jnp.float32)

    @pl.when(kf == 0)
    def _():
        acc_ref[...] = contrib

    @pl.when(kf > 0)
    def _():
        acc_ref[...] = acc_ref[...] + contrib

    @pl.when(kf == KF - 1)
    def _():
        eo = acc_ref[...] + eb2_ref[0]
        o_ref[...] = o_ref[...] + jnp.dot(
            oh * ts_ref[0], eo, precision=jax.lax.Precision.HIGHEST,
            preferred_element_type=jnp.float32)


def _experts(h3, ti, ts, ew1, eb1, ew2, eb2):
    return pl.pallas_call(
        _expert_kernel,
        grid=(E, KF),
        in_specs=[
            pl.BlockSpec((S, D), lambda e, kf: (0, 0)),
            pl.BlockSpec((1, 1, CAP), lambda e, kf: (e, 0, 0)),
            pl.BlockSpec((1, 1, CAP), lambda e, kf: (e, 0, 0)),
            pl.BlockSpec((1, D, DFB), lambda e, kf: (e, 0, kf)),
            pl.BlockSpec((1, 1, 1, DFB), lambda e, kf: (e, kf, 0, 0)),
            pl.BlockSpec((1, DFB, D), lambda e, kf: (e, kf, 0)),
            pl.BlockSpec((1, 1, D), lambda e, kf: (e, 0, 0)),
        ],
        out_specs=pl.BlockSpec((S, D), lambda e, kf: (0, 0)),
        out_shape=jax.ShapeDtypeStruct((S, D), jnp.float32),
        scratch_shapes=[
            pltpu.VMEM((CAP, D), jnp.float32),
            pltpu.VMEM((CAP, D), jnp.float32),
        ],
    )(
        h3,
        ti.reshape(E, 1, CAP),
        ts.reshape(E, 1, CAP),
        ew1,
        eb1.reshape(E, KF, 1, DFB),
        ew2,
        eb2.reshape(E, 1, D),
    )


def _add_kernel(a_ref, b_ref, o_ref):
    o_ref[...] = a_ref[...] + b_ref[...]


def _add(a, b, bm=256):
    M, N = a.shape
    return pl.pallas_call(
        _add_kernel,
        grid=(M // bm,),
        in_specs=[pl.BlockSpec((bm, N), lambda i: (i, 0))] * 2,
        out_specs=pl.BlockSpec((bm, N), lambda i: (i, 0)),
        out_shape=jax.ShapeDtypeStruct((M, N), jnp.float32),
    )(a, b)


def _rec_kernel(ev_ref, od_ref, cq_ref, cpw_ref, cpb_ref, it_ref, fm_ref, o_ref):
    inv = 1.0 / math.sqrt(D)
    ev = ev_ref[...]
    od = od_ref[...]
    cq = cq_ref[...]
    se = jnp.sum(ev * cq, axis=1, keepdims=True) * inv
    so = jnp.sum(od * cq, axis=1, keepdims=True) * inv
    m = jnp.maximum(se, so)
    ae = jnp.exp(se - m)
    ao = jnp.exp(so - m)
    mix = (ae * ev + ao * od) / (ae + ao)
    comp = jnp.dot(mix, cpw_ref[...], preferred_element_type=jnp.float32) + cpb_ref[...]
    dec = jnp.dot(it_ref[...], comp, preferred_element_type=jnp.float32)
    d = dec - fm_ref[...]
    ssq = jnp.sum(jnp.sum(d * d, axis=1, keepdims=True), axis=0, keepdims=True)
    o_ref[...] = ssq * (1.0 / (MEM_LEN * D))


def _interp_mat():
    L, out_len = MEM_LEN // 2, MEM_LEN
    pos = (np.arange(out_len, dtype=np.float64) + 0.5) * L / out_len - 0.5
    pos = np.clip(pos, 0.0, L - 1.0)
    lo = np.floor(pos).astype(np.int32)
    hi = np.clip(lo + 1, 0, L - 1)
    w = (pos - lo).astype(np.float32)
    mat = np.zeros((out_len, L), np.float32)
    mat[np.arange(out_len), lo] += 1.0 - w
    mat[np.arange(out_len), hi] += w
    return jnp.asarray(mat)


def _rec_loss(fine_mem, cq, cpw, cpb):
    fm3 = fine_mem.reshape(MEM_LEN // 2, 2, D)
    return pl.pallas_call(
        _rec_kernel,
        out_shape=jax.ShapeDtypeStruct((1, 1), jnp.float32),
    )(fm3[:, 0, :], fm3[:, 1, :], cq.reshape(1, D), cpw, cpb.reshape(1, D),
      _interp_mat(), fine_mem)[0, 0]


def _imp_kernel(s_ref, o_ref):
    x = s_ref[...]
    colid = jax.lax.broadcasted_iota(jnp.int32, x.shape, 1)
    valid = colid < E
    xm = jnp.where(valid, x, -1e30)
    m = jnp.max(xm, axis=1, keepdims=True)
    ex = jnp.where(valid, jnp.exp(x - m), 0.0)
    p = ex / jnp.sum(ex, axis=1, keepdims=True)
    imp = jnp.sum(p, axis=0, keepdims=True)
    mean = jnp.sum(imp, axis=1, keepdims=True) / E
    dv = jnp.where(valid[:1, :], imp - mean, 0.0)
    var = jnp.sum(dv * dv, axis=1, keepdims=True) / (E - 1)
    o_ref[...] = var / (mean * mean + 1e-6)


def _imp_loss(scores_pad):
    return pl.pallas_call(
        _imp_kernel,
        out_shape=jax.ShapeDtypeStruct((1, 1), jnp.float32),
    )(scores_pad)[0, 0]


def kernel(x, fine_mem, cmem, params):
    p = params
    xf = x.reshape(S, D)

    h = _ln(xf, p['g1'], p['b1'])
    wqkv = jnp.concatenate([p['Wq'], p['Wk'], p['Wv']], axis=1)
    qkv = _mm(h, wqkv)
    q = qkv[:, :D].reshape(S, H, HD).transpose(1, 0, 2)
    k = qkv[:, D:2 * D].reshape(S, H, HD).transpose(1, 0, 2)
    v = qkv[:, 2 * D:].reshape(S, H, HD).transpose(1, 0, 2)
    sc = _attn_scores(q, k)
    aw = jax.nn.softmax(sc, axis=-1)
    ao = _attn_av(aw, v).transpose(1, 0, 2).reshape(S, D)
    x1 = _mm(ao, p['Wo'], res=xf)

    fkv = _mm(fine_mem, jnp.concatenate([p['kpw'], p['vpw']], axis=1),
              jnp.concatenate([p['kpb'], p['vpb']]))
    ckv = _mm(cmem, jnp.concatenate([p['cmkw'], p['cmvw']], axis=1),
              jnp.concatenate([p['cmkb'], p['cmvb']]), bm=128)
    mem_k = jnp.concatenate([fkv[:, :D], ckv[:, :D]], axis=0)
    mem_v = jnp.concatenate([fkv[:, D:], ckv[:, D:]], axis=0)
    x2in = _ln(x1, p['g2'], p['b2'])
    qp = _mm(x2in, p['ckw'], p['ckb'])
    ms = _mm(qp, mem_k.T) / math.sqrt(D)
    ma = jax.nn.softmax(ms, axis=-1)
    mem_out = _mm(ma, mem_v, res=x2in, bn=512)
    x2 = _mm(mem_out, p['mpw'], p['mpb'], res=x1)

    rec = _rec_loss(fine_mem, p['cq'], p['cpw'], p['cpb'])

    h3 = _ln(x2, p['g3'], p['b3'])
    hr = _mm(h3, p['rw1'], p['rb1'], act="relu")
    rw2p = jnp.pad(p['rw2'], ((0, 0), (0, 128 - E)))
    rb2p = jnp.pad(p['rb2'], (0, 128 - E))
    scores_pad = _mm(hr, rw2p, rb2p, bn=128)
    scores = scores_pad[:, :E]
    ts, ti = jax.lax.top_k(scores.T, CAP)
    combined = _experts(h3, ti, ts, p['ew1'], p['eb1'], p['ew2'], p['eb2'])
    out = _add(x2, combined)

    imp = _imp_loss(scores_pad)
    aux = rec + imp
    return out.reshape(B, S, D), aux


# speedup vs baseline: 1.0703x; 1.0703x over previous
import functools
import math

import jax
import jax.numpy as jnp
import numpy as np
from jax.experimental import pallas as pl
from jax.experimental.pallas import tpu as pltpu

B, S, D = 1, 2048, 1024
H = 16
HD = D // H
E = 8
DFF = 4 * D
CAP = math.ceil(1.25 * S / E)
MEM_LEN = 256
CMEM_LEN = 128
KF = 8
DFB = DFF // KF


def _ln(x, g, b):
    m = jnp.mean(x, axis=-1, keepdims=True)
    v = jnp.mean((x - m) ** 2, axis=-1, keepdims=True)
    return (x - m) / jnp.sqrt(v + 1e-5) * g + b


def _mm(a, w, bias=None, *, act=None, res=None, bm=256, bn=512):
    M, K = a.shape
    N = w.shape[1]
    bm = min(bm, M)
    bn = min(bn, N)
    operands = [a, w]
    specs = [
        pl.BlockSpec((bm, K), lambda i, j: (i, 0)),
        pl.BlockSpec((K, bn), lambda i, j: (0, j)),
    ]
    has_bias = bias is not None
    has_res = res is not None
    if has_bias:
        operands.append(bias.reshape(1, N))
        specs.append(pl.BlockSpec((1, bn), lambda i, j: (0, j)))
    if has_res:
        operands.append(res)
        specs.append(pl.BlockSpec((bm, bn), lambda i, j: (i, j)))

    def kfn(*refs):
        it = iter(refs)
        a_ref = next(it)
        w_ref = next(it)
        b_ref = next(it) if has_bias else None
        r_ref = next(it) if has_res else None
        o_ref = next(it)
        o = jnp.dot(a_ref[...], w_ref[...], preferred_element_type=jnp.float32)
        if has_bias:
            o = o + b_ref[...]
        if act == "relu":
            o = jnp.maximum(o, 0.0)
        if has_res:
            o = o + r_ref[...]
        o_ref[...] = o

    return pl.pallas_call(
        kfn,
        grid=(M // bm, N // bn),
        in_specs=specs,
        out_specs=pl.BlockSpec((bm, bn), lambda i, j: (i, j)),
        out_shape=jax.ShapeDtypeStruct((M, N), jnp.float32),
    )(*operands)


def _attn_scores_kernel(q_ref, k_ref, o_ref, *, bq):
    i = pl.program_id(1)
    s = jax.lax.dot_general(
        q_ref[0], k_ref[0], (((1,), (1,)), ((), ())),
        preferred_element_type=jnp.float32,
    ) / math.sqrt(HD)
    row = i * bq + jax.lax.broadcasted_iota(jnp.int32, (bq, S), 0)
    col = jax.lax.broadcasted_iota(jnp.int32, (bq, S), 1)
    o_ref[0] = jnp.where(col > row, -1e30, s)


def _attn_scores(q, k, bq=512):
    return pl.pallas_call(
        functools.partial(_attn_scores_kernel, bq=bq),
        grid=(H, S // bq),
        in_specs=[
            pl.BlockSpec((1, bq, HD), lambda h, i: (h, i, 0)),
            pl.BlockSpec((1, S, HD), lambda h, i: (h, 0, 0)),
        ],
        out_specs=pl.BlockSpec((1, bq, S), lambda h, i: (h, i, 0)),
        out_shape=jax.ShapeDtypeStruct((H, S, S), jnp.float32),
    )(q, k)


def _attn_av_kernel(s_ref, l_ref, v_ref, o_ref):
    sv = s_ref[0]
    m = jnp.max(sv, axis=1, keepdims=True)
    e = jnp.exp(sv - m)
    w = e / jnp.transpose(l_ref[0, 0])
    o_ref[0] = jnp.dot(w, v_ref[0], preferred_element_type=jnp.float32)


def _attn_av(sc, l, v, bq=512):
    return pl.pallas_call(
        _attn_av_kernel,
        grid=(H, S // bq),
        in_specs=[
            pl.BlockSpec((1, bq, S), lambda h, i: (h, i, 0)),
            pl.BlockSpec((1, 1, 1, bq), lambda h, i: (h, i, 0, 0)),
            pl.BlockSpec((1, S, HD), lambda h, i: (h, 0, 0)),
        ],
        out_specs=pl.BlockSpec((1, bq, HD), lambda h, i: (h, i, 0)),
        out_shape=jax.ShapeDtypeStruct((H, S, HD), jnp.float32),
    )(sc, l.reshape(H, S // bq, 1, bq), v)


def _expert_kernel(h3_ref, ti_ref, ts_ref, ew1_ref, eb1_ref,
                   ew2_ref, eb2_ref, o_ref, disp_ref, acc_ref):
    e = pl.program_id(0)
    kf = pl.program_id(1)

    @pl.when(jnp.logical_and(e == 0, kf == 0))
    def _():
        o_ref[...] = jnp.zeros_like(o_ref)

    rows = jax.lax.broadcasted_iota(jnp.int32, (S, CAP), 0)
    oh = (rows == ti_ref[0]).astype(jnp.float32)

    @pl.when(kf == 0)
    def _():
        disp_ref[...] = jax.lax.dot_general(
            oh, h3_ref[...], (((0,), (0,)), ((), ())),
            precision=jax.lax.Precision.HIGHEST,
            preferred_element_type=jnp.float32,
        )

    h1 = jnp.maximum(
        jnp.dot(disp_ref[...], ew1_ref[0], preferred_element_type=jnp.float32)
        + eb1_ref[0, 0], 0.0)
    contrib = jnp.dot(h1, ew2_ref[0], preferred_element_type=jnp.float32)

    @pl.when(kf == 0)
    def _():
        acc_ref[...] = contrib

    @pl.when(kf > 0)
    def _():
        acc_ref[...] = acc_ref[...] + contrib

    @pl.when(kf == KF - 1)
    def _():
        eo = acc_ref[...] + eb2_ref[0]
        o_ref[...] = o_ref[...] + jnp.dot(
            oh * ts_ref[0], eo, precision=jax.lax.Precision.HIGHEST,
            preferred_element_type=jnp.float32)


def _experts(h3, ti, ts, ew1, eb1, ew2, eb2):
    return pl.pallas_call(
        _expert_kernel,
        grid=(E, KF),
        in_specs=[
            pl.BlockSpec((S, D), lambda e, kf: (0, 0)),
            pl.BlockSpec((1, 1, CAP), lambda e, kf: (e, 0, 0)),
            pl.BlockSpec((1, 1, CAP), lambda e, kf: (e, 0, 0)),
            pl.BlockSpec((1, D, DFB), lambda e, kf: (e, 0, kf)),
            pl.BlockSpec((1, 1, 1, DFB), lambda e, kf: (e, kf, 0, 0)),
            pl.BlockSpec((1, DFB, D), lambda e, kf: (e, kf, 0)),
            pl.BlockSpec((1, 1, D), lambda e, kf: (e, 0, 0)),
        ],
        out_specs=pl.BlockSpec((S, D), lambda e, kf: (0, 0)),
        out_shape=jax.ShapeDtypeStruct((S, D), jnp.float32),
        scratch_shapes=[
            pltpu.VMEM((CAP, D), jnp.float32),
            pltpu.VMEM((CAP, D), jnp.float32),
        ],
    )(
        h3,
        ti.reshape(E, 1, CAP),
        ts.reshape(E, 1, CAP),
        ew1,
        eb1.reshape(E, KF, 1, DFB),
        ew2,
        eb2.reshape(E, 1, D),
    )


def _add_kernel(a_ref, b_ref, o_ref):
    o_ref[...] = a_ref[...] + b_ref[...]


def _add(a, b, bm=256):
    M, N = a.shape
    return pl.pallas_call(
        _add_kernel,
        grid=(M // bm,),
        in_specs=[pl.BlockSpec((bm, N), lambda i: (i, 0))] * 2,
        out_specs=pl.BlockSpec((bm, N), lambda i: (i, 0)),
        out_shape=jax.ShapeDtypeStruct((M, N), jnp.float32),
    )(a, b)


def _rec_kernel(ev_ref, od_ref, cq_ref, cpw_ref, cpb_ref, it_ref, fm_ref, o_ref):
    inv = 1.0 / math.sqrt(D)
    ev = ev_ref[...]
    od = od_ref[...]
    cq = cq_ref[...]
    se = jnp.sum(ev * cq, axis=1, keepdims=True) * inv
    so = jnp.sum(od * cq, axis=1, keepdims=True) * inv
    m = jnp.maximum(se, so)
    ae = jnp.exp(se - m)
    ao = jnp.exp(so - m)
    mix = (ae * ev + ao * od) / (ae + ao)
    comp = jnp.dot(mix, cpw_ref[...], preferred_element_type=jnp.float32) + cpb_ref[...]
    dec = jnp.dot(it_ref[...], comp, preferred_element_type=jnp.float32)
    d = dec - fm_ref[...]
    ssq = jnp.sum(jnp.sum(d * d, axis=1, keepdims=True), axis=0, keepdims=True)
    o_ref[...] = ssq * (1.0 / (MEM_LEN * D))


def _interp_mat():
    L, out_len = MEM_LEN // 2, MEM_LEN
    pos = (np.arange(out_len, dtype=np.float64) + 0.5) * L / out_len - 0.5
    pos = np.clip(pos, 0.0, L - 1.0)
    lo = np.floor(pos).astype(np.int32)
    hi = np.clip(lo + 1, 0, L - 1)
    w = (pos - lo).astype(np.float32)
    mat = np.zeros((out_len, L), np.float32)
    mat[np.arange(out_len), lo] += 1.0 - w
    mat[np.arange(out_len), hi] += w
    return jnp.asarray(mat)


def _rec_loss(fine_mem, cq, cpw, cpb):
    fm3 = fine_mem.reshape(MEM_LEN // 2, 2, D)
    return pl.pallas_call(
        _rec_kernel,
        out_shape=jax.ShapeDtypeStruct((1, 1), jnp.float32),
    )(fm3[:, 0, :], fm3[:, 1, :], cq.reshape(1, D), cpw, cpb.reshape(1, D),
      _interp_mat(), fine_mem)[0, 0]


def _imp_kernel(s_ref, o_ref):
    x = s_ref[...]
    colid = jax.lax.broadcasted_iota(jnp.int32, x.shape, 1)
    valid = colid < E
    xm = jnp.where(valid, x, -1e30)
    m = jnp.max(xm, axis=1, keepdims=True)
    ex = jnp.where(valid, jnp.exp(x - m), 0.0)
    p = ex / jnp.sum(ex, axis=1, keepdims=True)
    imp = jnp.sum(p, axis=0, keepdims=True)
    mean = jnp.sum(imp, axis=1, keepdims=True) / E
    dv = jnp.where(valid[:1, :], imp - mean, 0.0)
    var = jnp.sum(dv * dv, axis=1, keepdims=True) / (E - 1)
    o_ref[...] = var / (mean * mean + 1e-6)


def _imp_loss(scores_pad):
    return pl.pallas_call(
        _imp_kernel,
        out_shape=jax.ShapeDtypeStruct((1, 1), jnp.float32),
    )(scores_pad)[0, 0]


def kernel(x, fine_mem, cmem, params):
    p = params
    xf = x.reshape(S, D)

    h = _ln(xf, p['g1'], p['b1'])
    wqkv = jnp.concatenate([p['Wq'], p['Wk'], p['Wv']], axis=1)
    qkv = _mm(h, wqkv)
    q = qkv[:, :D].reshape(S, H, HD).transpose(1, 0, 2)
    k = qkv[:, D:2 * D].reshape(S, H, HD).transpose(1, 0, 2)
    v = qkv[:, 2 * D:].reshape(S, H, HD).transpose(1, 0, 2)
    sc = _attn_scores(q, k)
    lsum = jnp.sum(jnp.exp(sc - jnp.max(sc, axis=-1, keepdims=True)), axis=-1)
    ao = _attn_av(sc, lsum, v).transpose(1, 0, 2).reshape(S, D)
    x1 = _mm(ao, p['Wo'], res=xf)

    fkv = _mm(fine_mem, jnp.concatenate([p['kpw'], p['vpw']], axis=1),
              jnp.concatenate([p['kpb'], p['vpb']]))
    ckv = _mm(cmem, jnp.concatenate([p['cmkw'], p['cmvw']], axis=1),
              jnp.concatenate([p['cmkb'], p['cmvb']]), bm=128)
    mem_k = jnp.concatenate([fkv[:, :D], ckv[:, :D]], axis=0)
    mem_v = jnp.concatenate([fkv[:, D:], ckv[:, D:]], axis=0)
    x2in = _ln(x1, p['g2'], p['b2'])
    qp = _mm(x2in, p['ckw'], p['ckb'])
    ms = _mm(qp, mem_k.T) / math.sqrt(D)
    ma = jax.nn.softmax(ms, axis=-1)
    mem_out = _mm(ma, mem_v, res=x2in, bn=512)
    x2 = _mm(mem_out, p['mpw'], p['mpb'], res=x1)

    rec = _rec_loss(fine_mem, p['cq'], p['cpw'], p['cpb'])

    h3 = _ln(x2, p['g3'], p['b3'])
    hr = _mm(h3, p['rw1'], p['rb1'], act="relu")
    rw2p = jnp.pad(p['rw2'], ((0, 0), (0, 128 - E)))
    rb2p = jnp.pad(p['rb2'], (0, 128 - E))
    scores_pad = _mm(hr, rw2p, rb2p, bn=128)
    scores = scores_pad[:, :E]
    ts, ti = jax.lax.top_k(scores.T, CAP)
    combined = _experts(h3, ti, ts, p['ew1'], p['eb1'], p['ew2'], p['eb2'])
    out = _add(x2, combined)

    imp = _imp_loss(scores_pad)
    aux = rec + imp
    return out.reshape(B, S, D), aux


# speedup vs baseline: 1.1308x; 1.0566x over previous
import functools
import math

import jax
import jax.numpy as jnp
import numpy as np
from jax.experimental import pallas as pl
from jax.experimental.pallas import tpu as pltpu
from jax.experimental.pallas import tpu_sc as plsc

B, S, D = 1, 2048, 1024
H = 16
HD = D // H
E = 8
DFF = 4 * D
CAP = math.ceil(1.25 * S / E)
MEM_LEN = 256
CMEM_LEN = 128
KF = 8
DFB = DFF // KF


def _ln(x, g, b):
    m = jnp.mean(x, axis=-1, keepdims=True)
    v = jnp.mean((x - m) ** 2, axis=-1, keepdims=True)
    return (x - m) / jnp.sqrt(v + 1e-5) * g + b


def _mm(a, w, bias=None, *, act=None, res=None, bm=256, bn=512):
    M, K = a.shape
    N = w.shape[1]
    bm = min(bm, M)
    bn = min(bn, N)
    operands = [a, w]
    specs = [
        pl.BlockSpec((bm, K), lambda i, j: (i, 0)),
        pl.BlockSpec((K, bn), lambda i, j: (0, j)),
    ]
    has_bias = bias is not None
    has_res = res is not None
    if has_bias:
        operands.append(bias.reshape(1, N))
        specs.append(pl.BlockSpec((1, bn), lambda i, j: (0, j)))
    if has_res:
        operands.append(res)
        specs.append(pl.BlockSpec((bm, bn), lambda i, j: (i, j)))

    def kfn(*refs):
        it = iter(refs)
        a_ref = next(it)
        w_ref = next(it)
        b_ref = next(it) if has_bias else None
        r_ref = next(it) if has_res else None
        o_ref = next(it)
        o = jnp.dot(a_ref[...], w_ref[...], preferred_element_type=jnp.float32)
        if has_bias:
            o = o + b_ref[...]
        if act == "relu":
            o = jnp.maximum(o, 0.0)
        if has_res:
            o = o + r_ref[...]
        o_ref[...] = o

    return pl.pallas_call(
        kfn,
        grid=(M // bm, N // bn),
        in_specs=specs,
        out_specs=pl.BlockSpec((bm, bn), lambda i, j: (i, j)),
        out_shape=jax.ShapeDtypeStruct((M, N), jnp.float32),
    )(*operands)


def _attn_scores_kernel(q_ref, k_ref, o_ref, *, bq):
    i = pl.program_id(1)
    s = jax.lax.dot_general(
        q_ref[0], k_ref[0], (((1,), (1,)), ((), ())),
        preferred_element_type=jnp.float32,
    ) / math.sqrt(HD)
    row = i * bq + jax.lax.broadcasted_iota(jnp.int32, (bq, S), 0)
    col = jax.lax.broadcasted_iota(jnp.int32, (bq, S), 1)
    o_ref[0] = jnp.where(col > row, -1e30, s)


def _attn_scores(q, k, bq=512):
    return pl.pallas_call(
        functools.partial(_attn_scores_kernel, bq=bq),
        grid=(H, S // bq),
        in_specs=[
            pl.BlockSpec((1, bq, HD), lambda h, i: (h, i, 0)),
            pl.BlockSpec((1, S, HD), lambda h, i: (h, 0, 0)),
        ],
        out_specs=pl.BlockSpec((1, bq, S), lambda h, i: (h, i, 0)),
        out_shape=jax.ShapeDtypeStruct((H, S, S), jnp.float32),
    )(q, k)


def _attn_av_kernel(s_ref, l_ref, v_ref, o_ref):
    sv = s_ref[0]
    m = jnp.max(sv, axis=1, keepdims=True)
    e = jnp.exp(sv - m)
    w = e / jnp.transpose(l_ref[0, 0])
    o_ref[0] = jnp.dot(w, v_ref[0], preferred_element_type=jnp.float32)


def _attn_av(sc, l, v, bq=512):
    return pl.pallas_call(
        _attn_av_kernel,
        grid=(H, S // bq),
        in_specs=[
            pl.BlockSpec((1, bq, S), lambda h, i: (h, i, 0)),
            pl.BlockSpec((1, 1, 1, bq), lambda h, i: (h, i, 0, 0)),
            pl.BlockSpec((1, S, HD), lambda h, i: (h, 0, 0)),
        ],
        out_specs=pl.BlockSpec((1, bq, HD), lambda h, i: (h, i, 0)),
        out_shape=jax.ShapeDtypeStruct((H, S, HD), jnp.float32),
    )(sc, l.reshape(H, S // bq, 1, bq), v)


def _sc_gather(table, idx):
    nidx = idx.shape[0]
    info = plsc.get_sparse_core_info()
    nw = info.num_cores * info.num_subcores
    b_per_w = nidx // nw
    mesh = plsc.VectorSubcoreMesh(core_axis_name="c", subcore_axis_name="s")

    @functools.partial(
        pl.kernel, mesh=mesh,
        out_type=jax.ShapeDtypeStruct((nidx, D), jnp.float32),
        scratch_types=[
            pltpu.VMEM((b_per_w,), jnp.int32),
            pltpu.VMEM((b_per_w, D), jnp.float32),
            pltpu.SemaphoreType.DMA,
        ],
    )
    def k(table_hbm, idx_hbm, out_hbm, idx_v, rows_v, sem):
        wid = jax.lax.axis_index("s") * info.num_cores + jax.lax.axis_index("c")
        base = wid * b_per_w
        pltpu.sync_copy(idx_hbm.at[pl.ds(base, b_per_w)], idx_v)
        pltpu.async_copy(table_hbm.at[idx_v], rows_v, sem).wait()
        pltpu.sync_copy(rows_v, out_hbm.at[pl.ds(base, b_per_w)])

    return k(table, idx)


def _expert_kernel(disp_ref, ti_ref, ts_ref, ew1_ref, eb1_ref,
                   ew2_ref, eb2_ref, o_ref, acc_ref):
    e = pl.program_id(0)
    kf = pl.program_id(1)

    @pl.when(jnp.logical_and(e == 0, kf == 0))
    def _():
        o_ref[...] = jnp.zeros_like(o_ref)

    rows = jax.lax.broadcasted_iota(jnp.int32, (S, CAP), 0)
    oh = (rows == ti_ref[0]).astype(jnp.float32)

    h1 = jnp.maximum(
        jnp.dot(disp_ref[0], ew1_ref[0], preferred_element_type=jnp.float32)
        + eb1_ref[0, 0], 0.0)
    contrib = jnp.dot(h1, ew2_ref[0], preferred_element_type=jnp.float32)

    @pl.when(kf == 0)
    def _():
        acc_ref[...] = contrib

    @pl.when(kf > 0)
    def _():
        acc_ref[...] = acc_ref[...] + contrib

    @pl.when(kf == KF - 1)
    def _():
        eo = acc_ref[...] + eb2_ref[0]
        o_ref[...] = o_ref[...] + jnp.dot(
            oh * ts_ref[0], eo, precision=jax.lax.Precision.HIGHEST,
            preferred_element_type=jnp.float32)


def _experts(disp, ti, ts, ew1, eb1, ew2, eb2):
    return pl.pallas_call(
        _expert_kernel,
        grid=(E, KF),
        in_specs=[
            pl.BlockSpec((1, CAP, D), lambda e, kf: (e, 0, 0)),
            pl.BlockSpec((1, 1, CAP), lambda e, kf: (e, 0, 0)),
            pl.BlockSpec((1, 1, CAP), lambda e, kf: (e, 0, 0)),
            pl.BlockSpec((1, D, DFB), lambda e, kf: (e, 0, kf)),
            pl.BlockSpec((1, 1, 1, DFB), lambda e, kf: (e, kf, 0, 0)),
            pl.BlockSpec((1, DFB, D), lambda e, kf: (e, kf, 0)),
            pl.BlockSpec((1, 1, D), lambda e, kf: (e, 0, 0)),
        ],
        out_specs=pl.BlockSpec((S, D), lambda e, kf: (0, 0)),
        out_shape=jax.ShapeDtypeStruct((S, D), jnp.float32),
        scratch_shapes=[
            pltpu.VMEM((CAP, D), jnp.float32),
        ],
    )(
        disp,
        ti.reshape(E, 1, CAP),
        ts.reshape(E, 1, CAP),
        ew1,
        eb1.reshape(E, KF, 1, DFB),
        ew2,
        eb2.reshape(E, 1, D),
    )


def _add_kernel(a_ref, b_ref, o_ref):
    o_ref[...] = a_ref[...] + b_ref[...]


def _add(a, b, bm=256):
    M, N = a.shape
    return pl.pallas_call(
        _add_kernel,
        grid=(M // bm,),
        in_specs=[pl.BlockSpec((bm, N), lambda i: (i, 0))] * 2,
        out_specs=pl.BlockSpec((bm, N), lambda i: (i, 0)),
        out_shape=jax.ShapeDtypeStruct((M, N), jnp.float32),
    )(a, b)


def _rec_kernel(ev_ref, od_ref, cq_ref, cpw_ref, cpb_ref, it_ref, fm_ref, o_ref):
    inv = 1.0 / math.sqrt(D)
    ev = ev_ref[...]
    od = od_ref[...]
    cq = cq_ref[...]
    se = jnp.sum(ev * cq, axis=1, keepdims=True) * inv
    so = jnp.sum(od * cq, axis=1, keepdims=True) * inv
    m = jnp.maximum(se, so)
    ae = jnp.exp(se - m)
    ao = jnp.exp(so - m)
    mix = (ae * ev + ao * od) / (ae + ao)
    comp = jnp.dot(mix, cpw_ref[...], preferred_element_type=jnp.float32) + cpb_ref[...]
    dec = jnp.dot(it_ref[...], comp, preferred_element_type=jnp.float32)
    d = dec - fm_ref[...]
    ssq = jnp.sum(jnp.sum(d * d, axis=1, keepdims=True), axis=0, keepdims=True)
    o_ref[...] = ssq * (1.0 / (MEM_LEN * D))


def _interp_mat():
    L, out_len = MEM_LEN // 2, MEM_LEN
    pos = (np.arange(out_len, dtype=np.float64) + 0.5) * L / out_len - 0.5
    pos = np.clip(pos, 0.0, L - 1.0)
    lo = np.floor(pos).astype(np.int32)
    hi = np.clip(lo + 1, 0, L - 1)
    w = (pos - lo).astype(np.float32)
    mat = np.zeros((out_len, L), np.float32)
    mat[np.arange(out_len), lo] += 1.0 - w
    mat[np.arange(out_len), hi] += w
    return jnp.asarray(mat)


def _rec_loss(fine_mem, cq, cpw, cpb):
    fm3 = fine_mem.reshape(MEM_LEN // 2, 2, D)
    return pl.pallas_call(
        _rec_kernel,
        out_shape=jax.ShapeDtypeStruct((1, 1), jnp.float32),
    )(fm3[:, 0, :], fm3[:, 1, :], cq.reshape(1, D), cpw, cpb.reshape(1, D),
      _interp_mat(), fine_mem)[0, 0]


def _imp_kernel(s_ref, o_ref):
    x = s_ref[...]
    colid = jax.lax.broadcasted_iota(jnp.int32, x.shape, 1)
    valid = colid < E
    xm = jnp.where(valid, x, -1e30)
    m = jnp.max(xm, axis=1, keepdims=True)
    ex = jnp.where(valid, jnp.exp(x - m), 0.0)
    p = ex / jnp.sum(ex, axis=1, keepdims=True)
    imp = jnp.sum(p, axis=0, keepdims=True)
    mean = jnp.sum(imp, axis=1, keepdims=True) / E
    dv = jnp.where(valid[:1, :], imp - mean, 0.0)
    var = jnp.sum(dv * dv, axis=1, keepdims=True) / (E - 1)
    o_ref[...] = var / (mean * mean + 1e-6)


def _imp_loss(scores_pad):
    return pl.pallas_call(
        _imp_kernel,
        out_shape=jax.ShapeDtypeStruct((1, 1), jnp.float32),
    )(scores_pad)[0, 0]


def kernel(x, fine_mem, cmem, params):
    p = params
    xf = x.reshape(S, D)

    h = _ln(xf, p['g1'], p['b1'])
    wqkv = jnp.concatenate([p['Wq'], p['Wk'], p['Wv']], axis=1)
    qkv = _mm(h, wqkv)
    q = qkv[:, :D].reshape(S, H, HD).transpose(1, 0, 2)
    k = qkv[:, D:2 * D].reshape(S, H, HD).transpose(1, 0, 2)
    v = qkv[:, 2 * D:].reshape(S, H, HD).transpose(1, 0, 2)
    sc = _attn_scores(q, k)
    lsum = jnp.sum(jnp.exp(sc - jnp.max(sc, axis=-1, keepdims=True)), axis=-1)
    ao = _attn_av(sc, lsum, v).transpose(1, 0, 2).reshape(S, D)
    x1 = _mm(ao, p['Wo'], res=xf)

    fkv = _mm(fine_mem, jnp.concatenate([p['kpw'], p['vpw']], axis=1),
              jnp.concatenate([p['kpb'], p['vpb']]))
    ckv = _mm(cmem, jnp.concatenate([p['cmkw'], p['cmvw']], axis=1),
              jnp.concatenate([p['cmkb'], p['cmvb']]), bm=128)
    mem_k = jnp.concatenate([fkv[:, :D], ckv[:, :D]], axis=0)
    mem_v = jnp.concatenate([fkv[:, D:], ckv[:, D:]], axis=0)
    x2in = _ln(x1, p['g2'], p['b2'])
    qp = _mm(x2in, p['ckw'], p['ckb'])
    ms = _mm(qp, mem_k.T) / math.sqrt(D)
    ma = jax.nn.softmax(ms, axis=-1)
    mem_out = _mm(ma, mem_v, res=x2in, bn=512)
    x2 = _mm(mem_out, p['mpw'], p['mpb'], res=x1)

    rec = _rec_loss(fine_mem, p['cq'], p['cpw'], p['cpb'])

    h3 = _ln(x2, p['g3'], p['b3'])
    hr = _mm(h3, p['rw1'], p['rb1'], act="relu")
    rw2p = jnp.pad(p['rw2'], ((0, 0), (0, 128 - E)))
    rb2p = jnp.pad(p['rb2'], (0, 128 - E))
    scores_pad = _mm(hr, rw2p, rb2p, bn=128)
    scores = scores_pad[:, :E]
    ts, ti = jax.lax.top_k(scores.T, CAP)
    disp = _sc_gather(h3, ti.reshape(-1)).reshape(E, CAP, D)
    combined = _experts(disp, ti, ts, p['ew1'], p['eb1'], p['ew2'], p['eb2'])
    out = _add(x2, combined)

    imp = _imp_loss(scores_pad)
    aux = rec + imp
    return out.reshape(B, S, D), aux


# speedup vs baseline: 1.1499x; 1.0168x over previous
import functools
import math

import jax
import jax.numpy as jnp
import numpy as np
from jax.experimental import pallas as pl
from jax.experimental.pallas import tpu as pltpu
from jax.experimental.pallas import tpu_sc as plsc

B, S, D = 1, 2048, 1024
H = 16
HD = D // H
E = 8
DFF = 4 * D
CAP = math.ceil(1.25 * S / E)
MEM_LEN = 256
CMEM_LEN = 128
KF = 8
DFB = DFF // KF


def _ln(x, g, b):
    m = jnp.mean(x, axis=-1, keepdims=True)
    v = jnp.mean((x - m) ** 2, axis=-1, keepdims=True)
    return (x - m) / jnp.sqrt(v + 1e-5) * g + b


def _mm(a, w, bias=None, *, act=None, res=None, bm=256, bn=512):
    M, K = a.shape
    N = w.shape[1]
    bm = min(bm, M)
    bn = min(bn, N)
    operands = [a, w]
    specs = [
        pl.BlockSpec((bm, K), lambda i, j: (i, 0)),
        pl.BlockSpec((K, bn), lambda i, j: (0, j)),
    ]
    has_bias = bias is not None
    has_res = res is not None
    if has_bias:
        operands.append(bias.reshape(1, N))
        specs.append(pl.BlockSpec((1, bn), lambda i, j: (0, j)))
    if has_res:
        operands.append(res)
        specs.append(pl.BlockSpec((bm, bn), lambda i, j: (i, j)))

    def kfn(*refs):
        it = iter(refs)
        a_ref = next(it)
        w_ref = next(it)
        b_ref = next(it) if has_bias else None
        r_ref = next(it) if has_res else None
        o_ref = next(it)
        o = jnp.dot(a_ref[...], w_ref[...], preferred_element_type=jnp.float32)
        if has_bias:
            o = o + b_ref[...]
        if act == "relu":
            o = jnp.maximum(o, 0.0)
        if has_res:
            o = o + r_ref[...]
        o_ref[...] = o

    return pl.pallas_call(
        kfn,
        grid=(M // bm, N // bn),
        in_specs=specs,
        out_specs=pl.BlockSpec((bm, bn), lambda i, j: (i, j)),
        out_shape=jax.ShapeDtypeStruct((M, N), jnp.float32),
    )(*operands)


def _attn_scores_kernel(q_ref, k_ref, o_ref, *, bq):
    i = pl.program_id(1)
    row = i * bq + jax.lax.broadcasted_iota(jnp.int32, (bq, S), 0)
    col = jax.lax.broadcasted_iota(jnp.int32, (bq, S), 1)
    for ii in range(S // bq):
        @pl.when(i == ii)
        def _(ii=ii):
            w = (ii + 1) * bq
            s = jax.lax.dot_general(
                q_ref[0], k_ref[0, :w, :], (((1,), (1,)), ((), ())),
                preferred_element_type=jnp.float32,
            ) / math.sqrt(HD)
            if w < S:
                s = jnp.concatenate(
                    [s, jnp.full((bq, S - w), -1e30, jnp.float32)], axis=1)
            o_ref[0] = jnp.where(col > row, -1e30, s)


def _attn_scores(q, k, bq=512):
    return pl.pallas_call(
        functools.partial(_attn_scores_kernel, bq=bq),
        grid=(H, S // bq),
        in_specs=[
            pl.BlockSpec((1, bq, HD), lambda h, i: (h, i, 0)),
            pl.BlockSpec((1, S, HD), lambda h, i: (h, 0, 0)),
        ],
        out_specs=pl.BlockSpec((1, bq, S), lambda h, i: (h, i, 0)),
        out_shape=jax.ShapeDtypeStruct((H, S, S), jnp.float32),
    )(q, k)


def _attn_av_kernel(s_ref, l_ref, v_ref, o_ref, *, bq):
    i = pl.program_id(1)
    for ii in range(S // bq):
        @pl.when(i == ii)
        def _(ii=ii):
            wd = (ii + 1) * bq
            sv = s_ref[0, :, :wd]
            m = jnp.max(sv, axis=1, keepdims=True)
            e = jnp.exp(sv - m)
            w = e / jnp.transpose(l_ref[0, 0])
            o_ref[0] = jnp.dot(w, v_ref[0, :wd, :],
                               preferred_element_type=jnp.float32)


def _attn_av(sc, l, v, bq=512):
    return pl.pallas_call(
        functools.partial(_attn_av_kernel, bq=bq),
        grid=(H, S // bq),
        in_specs=[
            pl.BlockSpec((1, bq, S), lambda h, i: (h, i, 0)),
            pl.BlockSpec((1, 1, 1, bq), lambda h, i: (h, i, 0, 0)),
            pl.BlockSpec((1, S, HD), lambda h, i: (h, 0, 0)),
        ],
        out_specs=pl.BlockSpec((1, bq, HD), lambda h, i: (h, i, 0)),
        out_shape=jax.ShapeDtypeStruct((H, S, HD), jnp.float32),
    )(sc, l.reshape(H, S // bq, 1, bq), v)


def _sc_gather(table, idx):
    nidx = idx.shape[0]
    info = plsc.get_sparse_core_info()
    nw = info.num_cores * info.num_subcores
    b_per_w = nidx // nw
    mesh = plsc.VectorSubcoreMesh(core_axis_name="c", subcore_axis_name="s")

    @functools.partial(
        pl.kernel, mesh=mesh,
        out_type=jax.ShapeDtypeStruct((nidx, D), jnp.float32),
        scratch_types=[
            pltpu.VMEM((b_per_w,), jnp.int32),
            pltpu.VMEM((b_per_w, D), jnp.float32),
            pltpu.SemaphoreType.DMA,
        ],
    )
    def k(table_hbm, idx_hbm, out_hbm, idx_v, rows_v, sem):
        wid = jax.lax.axis_index("s") * info.num_cores + jax.lax.axis_index("c")
        base = wid * b_per_w
        pltpu.sync_copy(idx_hbm.at[pl.ds(base, b_per_w)], idx_v)
        pltpu.async_copy(table_hbm.at[idx_v], rows_v, sem).wait()
        pltpu.sync_copy(rows_v, out_hbm.at[pl.ds(base, b_per_w)])

    return k(table, idx)


def _expert_kernel(disp_ref, ti_ref, ts_ref, ew1_ref, eb1_ref,
                   ew2_ref, eb2_ref, o_ref, acc_ref):
    e = pl.program_id(0)
    kf = pl.program_id(1)

    @pl.when(jnp.logical_and(e == 0, kf == 0))
    def _():
        o_ref[...] = jnp.zeros_like(o_ref)

    rows = jax.lax.broadcasted_iota(jnp.int32, (S, CAP), 0)
    oh = (rows == ti_ref[0]).astype(jnp.float32)

    h1 = jnp.maximum(
        jnp.dot(disp_ref[0], ew1_ref[0], preferred_element_type=jnp.float32)
        + eb1_ref[0, 0], 0.0)
    contrib = jnp.dot(h1, ew2_ref[0], preferred_element_type=jnp.float32)

    @pl.when(kf == 0)
    def _():
        acc_ref[...] = contrib

    @pl.when(kf > 0)
    def _():
        acc_ref[...] = acc_ref[...] + contrib

    @pl.when(kf == KF - 1)
    def _():
        eo = acc_ref[...] + eb2_ref[0]
        o_ref[...] = o_ref[...] + jnp.dot(
            oh * ts_ref[0], eo, precision=jax.lax.Precision.HIGHEST,
            preferred_element_type=jnp.float32)


def _experts(disp, ti, ts, ew1, eb1, ew2, eb2):
    return pl.pallas_call(
        _expert_kernel,
        grid=(E, KF),
        in_specs=[
            pl.BlockSpec((1, CAP, D), lambda e, kf: (e, 0, 0)),
            pl.BlockSpec((1, 1, CAP), lambda e, kf: (e, 0, 0)),
            pl.BlockSpec((1, 1, CAP), lambda e, kf: (e, 0, 0)),
            pl.BlockSpec((1, D, DFB), lambda e, kf: (e, 0, kf)),
            pl.BlockSpec((1, 1, 1, DFB), lambda e, kf: (e, kf, 0, 0)),
            pl.BlockSpec((1, DFB, D), lambda e, kf: (e, kf, 0)),
            pl.BlockSpec((1, 1, D), lambda e, kf: (e, 0, 0)),
        ],
        out_specs=pl.BlockSpec((S, D), lambda e, kf: (0, 0)),
        out_shape=jax.ShapeDtypeStruct((S, D), jnp.float32),
        scratch_shapes=[
            pltpu.VMEM((CAP, D), jnp.float32),
        ],
    )(
        disp,
        ti.reshape(E, 1, CAP),
        ts.reshape(E, 1, CAP),
        ew1,
        eb1.reshape(E, KF, 1, DFB),
        ew2,
        eb2.reshape(E, 1, D),
    )


def _add_kernel(a_ref, b_ref, o_ref):
    o_ref[...] = a_ref[...] + b_ref[...]


def _add(a, b, bm=256):
    M, N = a.shape
    return pl.pallas_call(
        _add_kernel,
        grid=(M // bm,),
        in_specs=[pl.BlockSpec((bm, N), lambda i: (i, 0))] * 2,
        out_specs=pl.BlockSpec((bm, N), lambda i: (i, 0)),
        out_shape=jax.ShapeDtypeStruct((M, N), jnp.float32),
    )(a, b)


def _rec_kernel(ev_ref, od_ref, cq_ref, cpw_ref, cpb_ref, it_ref, fm_ref, o_ref):
    inv = 1.0 / math.sqrt(D)
    ev = ev_ref[...]
    od = od_ref[...]
    cq = cq_ref[...]
    se = jnp.sum(ev * cq, axis=1, keepdims=True) * inv
    so = jnp.sum(od * cq, axis=1, keepdims=True) * inv
    m = jnp.maximum(se, so)
    ae = jnp.exp(se - m)
    ao = jnp.exp(so - m)
    mix = (ae * ev + ao * od) / (ae + ao)
    comp = jnp.dot(mix, cpw_ref[...], preferred_element_type=jnp.float32) + cpb_ref[...]
    dec = jnp.dot(it_ref[...], comp, preferred_element_type=jnp.float32)
    d = dec - fm_ref[...]
    ssq = jnp.sum(jnp.sum(d * d, axis=1, keepdims=True), axis=0, keepdims=True)
    o_ref[...] = ssq * (1.0 / (MEM_LEN * D))


def _interp_mat():
    L, out_len = MEM_LEN // 2, MEM_LEN
    pos = (np.arange(out_len, dtype=np.float64) + 0.5) * L / out_len - 0.5
    pos = np.clip(pos, 0.0, L - 1.0)
    lo = np.floor(pos).astype(np.int32)
    hi = np.clip(lo + 1, 0, L - 1)
    w = (pos - lo).astype(np.float32)
    mat = np.zeros((out_len, L), np.float32)
    mat[np.arange(out_len), lo] += 1.0 - w
    mat[np.arange(out_len), hi] += w
    return jnp.asarray(mat)


def _rec_loss(fine_mem, cq, cpw, cpb):
    fm3 = fine_mem.reshape(MEM_LEN // 2, 2, D)
    return pl.pallas_call(
        _rec_kernel,
        out_shape=jax.ShapeDtypeStruct((1, 1), jnp.float32),
    )(fm3[:, 0, :], fm3[:, 1, :], cq.reshape(1, D), cpw, cpb.reshape(1, D),
      _interp_mat(), fine_mem)[0, 0]


def _imp_kernel(s_ref, o_ref):
    x = s_ref[...]
    colid = jax.lax.broadcasted_iota(jnp.int32, x.shape, 1)
    valid = colid < E
    xm = jnp.where(valid, x, -1e30)
    m = jnp.max(xm, axis=1, keepdims=True)
    ex = jnp.where(valid, jnp.exp(x - m), 0.0)
    p = ex / jnp.sum(ex, axis=1, keepdims=True)
    imp = jnp.sum(p, axis=0, keepdims=True)
    mean = jnp.sum(imp, axis=1, keepdims=True) / E
    dv = jnp.where(valid[:1, :], imp - mean, 0.0)
    var = jnp.sum(dv * dv, axis=1, keepdims=True) / (E - 1)
    o_ref[...] = var / (mean * mean + 1e-6)


def _imp_loss(scores_pad):
    return pl.pallas_call(
        _imp_kernel,
        out_shape=jax.ShapeDtypeStruct((1, 1), jnp.float32),
    )(scores_pad)[0, 0]


def kernel(x, fine_mem, cmem, params):
    p = params
    xf = x.reshape(S, D)

    h = _ln(xf, p['g1'], p['b1'])
    wqkv = jnp.concatenate([p['Wq'], p['Wk'], p['Wv']], axis=1)
    qkv = _mm(h, wqkv)
    q = qkv[:, :D].reshape(S, H, HD).transpose(1, 0, 2)
    k = qkv[:, D:2 * D].reshape(S, H, HD).transpose(1, 0, 2)
    v = qkv[:, 2 * D:].reshape(S, H, HD).transpose(1, 0, 2)
    sc = _attn_scores(q, k)
    lsum = jnp.sum(jnp.exp(sc - jnp.max(sc, axis=-1, keepdims=True)), axis=-1)
    ao = _attn_av(sc, lsum, v).transpose(1, 0, 2).reshape(S, D)
    x1 = _mm(ao, p['Wo'], res=xf)

    fkv = _mm(fine_mem, jnp.concatenate([p['kpw'], p['vpw']], axis=1),
              jnp.concatenate([p['kpb'], p['vpb']]))
    ckv = _mm(cmem, jnp.concatenate([p['cmkw'], p['cmvw']], axis=1),
              jnp.concatenate([p['cmkb'], p['cmvb']]), bm=128)
    mem_k = jnp.concatenate([fkv[:, :D], ckv[:, :D]], axis=0)
    mem_v = jnp.concatenate([fkv[:, D:], ckv[:, D:]], axis=0)
    x2in = _ln(x1, p['g2'], p['b2'])
    qp = _mm(x2in, p['ckw'], p['ckb'])
    ms = _mm(qp, mem_k.T) / math.sqrt(D)
    ma = jax.nn.softmax(ms, axis=-1)
    mem_out = _mm(ma, mem_v, res=x2in, bn=512)
    x2 = _mm(mem_out, p['mpw'], p['mpb'], res=x1)

    rec = _rec_loss(fine_mem, p['cq'], p['cpw'], p['cpb'])

    h3 = _ln(x2, p['g3'], p['b3'])
    hr = _mm(h3, p['rw1'], p['rb1'], act="relu")
    rw2p = jnp.pad(p['rw2'], ((0, 0), (0, 128 - E)))
    rb2p = jnp.pad(p['rb2'], (0, 128 - E))
    scores_pad = _mm(hr, rw2p, rb2p, bn=128)
    scores = scores_pad[:, :E]
    ts, ti = jax.lax.top_k(scores.T, CAP)
    disp = _sc_gather(h3, ti.reshape(-1)).reshape(E, CAP, D)
    combined = _experts(disp, ti, ts, p['ew1'], p['eb1'], p['ew2'], p['eb2'])
    out = _add(x2, combined)

    imp = _imp_loss(scores_pad)
    aux = rec + imp
    return out.reshape(B, S, D), aux


# speedup vs baseline: 1.2496x; 1.0867x over previous
import functools
import math

import jax
import jax.numpy as jnp
import numpy as np
from jax.experimental import pallas as pl
from jax.experimental.pallas import tpu as pltpu
from jax.experimental.pallas import tpu_sc as plsc

B, S, D = 1, 2048, 1024
H = 16
HD = D // H
E = 8
DFF = 4 * D
CAP = math.ceil(1.25 * S / E)
MEM_LEN = 256
CMEM_LEN = 128
KF = 8
DFB = DFF // KF


def _ln(x, g, b):
    m = jnp.mean(x, axis=-1, keepdims=True)
    v = jnp.mean((x - m) ** 2, axis=-1, keepdims=True)
    return (x - m) / jnp.sqrt(v + 1e-5) * g + b


def _mm(a, w, bias=None, *, act=None, res=None, bm=256, bn=512):
    M, K = a.shape
    N = w.shape[1]
    bm = min(bm, M)
    bn = min(bn, N)
    operands = [a, w]
    specs = [
        pl.BlockSpec((bm, K), lambda i, j: (i, 0)),
        pl.BlockSpec((K, bn), lambda i, j: (0, j)),
    ]
    has_bias = bias is not None
    has_res = res is not None
    if has_bias:
        operands.append(bias.reshape(1, N))
        specs.append(pl.BlockSpec((1, bn), lambda i, j: (0, j)))
    if has_res:
        operands.append(res)
        specs.append(pl.BlockSpec((bm, bn), lambda i, j: (i, j)))

    def kfn(*refs):
        it = iter(refs)
        a_ref = next(it)
        w_ref = next(it)
        b_ref = next(it) if has_bias else None
        r_ref = next(it) if has_res else None
        o_ref = next(it)
        o = jnp.dot(a_ref[...], w_ref[...], preferred_element_type=jnp.float32)
        if has_bias:
            o = o + b_ref[...]
        if act == "relu":
            o = jnp.maximum(o, 0.0)
        if has_res:
            o = o + r_ref[...]
        o_ref[...] = o

    return pl.pallas_call(
        kfn,
        grid=(M // bm, N // bn),
        in_specs=specs,
        out_specs=pl.BlockSpec((bm, bn), lambda i, j: (i, j)),
        out_shape=jax.ShapeDtypeStruct((M, N), jnp.float32),
    )(*operands)


def _attn_scores_kernel(q_ref, k_ref, o_ref, *, bq):
    i = pl.program_id(1)
    row = i * bq + jax.lax.broadcasted_iota(jnp.int32, (bq, S), 0)
    col = jax.lax.broadcasted_iota(jnp.int32, (bq, S), 1)
    for ii in range(S // bq):
        @pl.when(i == ii)
        def _(ii=ii):
            w = (ii + 1) * bq
            s = jax.lax.dot_general(
                q_ref[0], k_ref[0, :w, :], (((1,), (1,)), ((), ())),
                preferred_element_type=jnp.float32,
            ) / math.sqrt(HD)
            if w < S:
                s = jnp.concatenate(
                    [s, jnp.full((bq, S - w), -1e30, jnp.float32)], axis=1)
            o_ref[0] = jnp.where(col > row, -1e30, s)


def _attn_scores(q, k, bq=512):
    return pl.pallas_call(
        functools.partial(_attn_scores_kernel, bq=bq),
        grid=(H, S // bq),
        in_specs=[
            pl.BlockSpec((1, bq, HD), lambda h, i: (h, i, 0)),
            pl.BlockSpec((1, S, HD), lambda h, i: (h, 0, 0)),
        ],
        out_specs=pl.BlockSpec((1, bq, S), lambda h, i: (h, i, 0)),
        out_shape=jax.ShapeDtypeStruct((H, S, S), jnp.float32),
    )(q, k)


def _attn_av_kernel(s_ref, l_ref, v_ref, o_ref, *, bq):
    i = pl.program_id(1)
    for ii in range(S // bq):
        @pl.when(i == ii)
        def _(ii=ii):
            wd = (ii + 1) * bq
            sv = s_ref[0, :, :wd]
            m = jnp.max(sv, axis=1, keepdims=True)
            e = jnp.exp(sv - m)
            w = e / jnp.transpose(l_ref[0, 0])
            o_ref[0] = jnp.dot(w, v_ref[0, :wd, :],
                               preferred_element_type=jnp.float32)


def _attn_av(sc, l, v, bq=512):
    return pl.pallas_call(
        functools.partial(_attn_av_kernel, bq=bq),
        grid=(H, S // bq),
        in_specs=[
            pl.BlockSpec((1, bq, S), lambda h, i: (h, i, 0)),
            pl.BlockSpec((1, 1, 1, bq), lambda h, i: (h, i, 0, 0)),
            pl.BlockSpec((1, S, HD), lambda h, i: (h, 0, 0)),
        ],
        out_specs=pl.BlockSpec((1, bq, HD), lambda h, i: (h, i, 0)),
        out_shape=jax.ShapeDtypeStruct((H, S, HD), jnp.float32),
    )(sc, l.reshape(H, S // bq, 1, bq), v)


def _sc_gather(table, idx):
    nidx = idx.shape[0]
    info = plsc.get_sparse_core_info()
    nw = info.num_cores * info.num_subcores
    b_per_w = nidx // nw
    mesh = plsc.VectorSubcoreMesh(core_axis_name="c", subcore_axis_name="s")

    @functools.partial(
        pl.kernel, mesh=mesh,
        out_type=jax.ShapeDtypeStruct((nidx, D), jnp.float32),
        scratch_types=[
            pltpu.VMEM((b_per_w,), jnp.int32),
            pltpu.VMEM((b_per_w, D), jnp.float32),
            pltpu.SemaphoreType.DMA,
        ],
    )
    def k(table_hbm, idx_hbm, out_hbm, idx_v, rows_v, sem):
        wid = jax.lax.axis_index("s") * info.num_cores + jax.lax.axis_index("c")
        base = wid * b_per_w
        pltpu.sync_copy(idx_hbm.at[pl.ds(base, b_per_w)], idx_v)
        pltpu.async_copy(table_hbm.at[idx_v], rows_v, sem).wait()
        pltpu.sync_copy(rows_v, out_hbm.at[pl.ds(base, b_per_w)])

    return k(table, idx)


def _expert_kernel(disp_ref, ti_ref, ts_ref, ew1_ref, eb1_ref,
                   ew2_ref, eb2_ref, o_ref, acc_ref):
    e = pl.program_id(0)
    kf = pl.program_id(1)

    @pl.when(jnp.logical_and(e == 0, kf == 0))
    def _():
        o_ref[...] = jnp.zeros_like(o_ref)

    rows = jax.lax.broadcasted_iota(jnp.int32, (S, CAP), 0)
    oh = (rows == ti_ref[0]).astype(jnp.float32)

    h1 = jnp.maximum(
        jnp.dot(disp_ref[0], ew1_ref[0], preferred_element_type=jnp.float32)
        + eb1_ref[0, 0], 0.0)
    contrib = jnp.dot(h1, ew2_ref[0], preferred_element_type=jnp.float32)

    @pl.when(kf == 0)
    def _():
        acc_ref[...] = contrib

    @pl.when(kf > 0)
    def _():
        acc_ref[...] = acc_ref[...] + contrib

    @pl.when(kf == KF - 1)
    def _():
        eo = acc_ref[...] + eb2_ref[0]
        o_ref[...] = o_ref[...] + jnp.dot(
            oh * ts_ref[0], eo, preferred_element_type=jnp.float32)


def _experts(disp, ti, ts, ew1, eb1, ew2, eb2):
    return pl.pallas_call(
        _expert_kernel,
        grid=(E, KF),
        in_specs=[
            pl.BlockSpec((1, CAP, D), lambda e, kf: (e, 0, 0)),
            pl.BlockSpec((1, 1, CAP), lambda e, kf: (e, 0, 0)),
            pl.BlockSpec((1, 1, CAP), lambda e, kf: (e, 0, 0)),
            pl.BlockSpec((1, D, DFB), lambda e, kf: (e, 0, kf)),
            pl.BlockSpec((1, 1, 1, DFB), lambda e, kf: (e, kf, 0, 0)),
            pl.BlockSpec((1, DFB, D), lambda e, kf: (e, kf, 0)),
            pl.BlockSpec((1, 1, D), lambda e, kf: (e, 0, 0)),
        ],
        out_specs=pl.BlockSpec((S, D), lambda e, kf: (0, 0)),
        out_shape=jax.ShapeDtypeStruct((S, D), jnp.float32),
        scratch_shapes=[
            pltpu.VMEM((CAP, D), jnp.float32),
        ],
    )(
        disp,
        ti.reshape(E, 1, CAP),
        ts.reshape(E, 1, CAP),
        ew1,
        eb1.reshape(E, KF, 1, DFB),
        ew2,
        eb2.reshape(E, 1, D),
    )


def _add_kernel(a_ref, b_ref, o_ref):
    o_ref[...] = a_ref[...] + b_ref[...]


def _add(a, b, bm=256):
    M, N = a.shape
    return pl.pallas_call(
        _add_kernel,
        grid=(M // bm,),
        in_specs=[pl.BlockSpec((bm, N), lambda i: (i, 0))] * 2,
        out_specs=pl.BlockSpec((bm, N), lambda i: (i, 0)),
        out_shape=jax.ShapeDtypeStruct((M, N), jnp.float32),
    )(a, b)


def _rec_kernel(ev_ref, od_ref, cq_ref, cpw_ref, cpb_ref, it_ref, fm_ref, o_ref):
    inv = 1.0 / math.sqrt(D)
    ev = ev_ref[...]
    od = od_ref[...]
    cq = cq_ref[...]
    se = jnp.sum(ev * cq, axis=1, keepdims=True) * inv
    so = jnp.sum(od * cq, axis=1, keepdims=True) * inv
    m = jnp.maximum(se, so)
    ae = jnp.exp(se - m)
    ao = jnp.exp(so - m)
    mix = (ae * ev + ao * od) / (ae + ao)
    comp = jnp.dot(mix, cpw_ref[...], preferred_element_type=jnp.float32) + cpb_ref[...]
    dec = jnp.dot(it_ref[...], comp, preferred_element_type=jnp.float32)
    d = dec - fm_ref[...]
    ssq = jnp.sum(jnp.sum(d * d, axis=1, keepdims=True), axis=0, keepdims=True)
    o_ref[...] = ssq * (1.0 / (MEM_LEN * D))


def _interp_mat():
    L, out_len = MEM_LEN // 2, MEM_LEN
    pos = (np.arange(out_len, dtype=np.float64) + 0.5) * L / out_len - 0.5
    pos = np.clip(pos, 0.0, L - 1.0)
    lo = np.floor(pos).astype(np.int32)
    hi = np.clip(lo + 1, 0, L - 1)
    w = (pos - lo).astype(np.float32)
    mat = np.zeros((out_len, L), np.float32)
    mat[np.arange(out_len), lo] += 1.0 - w
    mat[np.arange(out_len), hi] += w
    return jnp.asarray(mat)


def _rec_loss(fine_mem, cq, cpw, cpb):
    fm3 = fine_mem.reshape(MEM_LEN // 2, 2, D)
    return pl.pallas_call(
        _rec_kernel,
        out_shape=jax.ShapeDtypeStruct((1, 1), jnp.float32),
    )(fm3[:, 0, :], fm3[:, 1, :], cq.reshape(1, D), cpw, cpb.reshape(1, D),
      _interp_mat(), fine_mem)[0, 0]


def _imp_kernel(s_ref, o_ref):
    x = s_ref[...]
    colid = jax.lax.broadcasted_iota(jnp.int32, x.shape, 1)
    valid = colid < E
    xm = jnp.where(valid, x, -1e30)
    m = jnp.max(xm, axis=1, keepdims=True)
    ex = jnp.where(valid, jnp.exp(x - m), 0.0)
    p = ex / jnp.sum(ex, axis=1, keepdims=True)
    imp = jnp.sum(p, axis=0, keepdims=True)
    mean = jnp.sum(imp, axis=1, keepdims=True) / E
    dv = jnp.where(valid[:1, :], imp - mean, 0.0)
    var = jnp.sum(dv * dv, axis=1, keepdims=True) / (E - 1)
    o_ref[...] = var / (mean * mean + 1e-6)


def _imp_loss(scores_pad):
    return pl.pallas_call(
        _imp_kernel,
        out_shape=jax.ShapeDtypeStruct((1, 1), jnp.float32),
    )(scores_pad)[0, 0]


def kernel(x, fine_mem, cmem, params):
    p = params
    xf = x.reshape(S, D)

    h = _ln(xf, p['g1'], p['b1'])
    wqkv = jnp.concatenate([p['Wq'], p['Wk'], p['Wv']], axis=1)
    qkv = _mm(h, wqkv)
    q = qkv[:, :D].reshape(S, H, HD).transpose(1, 0, 2)
    k = qkv[:, D:2 * D].reshape(S, H, HD).transpose(1, 0, 2)
    v = qkv[:, 2 * D:].reshape(S, H, HD).transpose(1, 0, 2)
    sc = _attn_scores(q, k)
    lsum = jnp.sum(jnp.exp(sc - jnp.max(sc, axis=-1, keepdims=True)), axis=-1)
    ao = _attn_av(sc, lsum, v).transpose(1, 0, 2).reshape(S, D)
    x1 = _mm(ao, p['Wo'], res=xf)

    fkv = _mm(fine_mem, jnp.concatenate([p['kpw'], p['vpw']], axis=1),
              jnp.concatenate([p['kpb'], p['vpb']]))
    ckv = _mm(cmem, jnp.concatenate([p['cmkw'], p['cmvw']], axis=1),
              jnp.concatenate([p['cmkb'], p['cmvb']]), bm=128)
    mem_k = jnp.concatenate([fkv[:, :D], ckv[:, :D]], axis=0)
    mem_v = jnp.concatenate([fkv[:, D:], ckv[:, D:]], axis=0)
    x2in = _ln(x1, p['g2'], p['b2'])
    qp = _mm(x2in, p['ckw'], p['ckb'])
    ms = _mm(qp, mem_k.T) / math.sqrt(D)
    ma = jax.nn.softmax(ms, axis=-1)
    mem_out = _mm(ma, mem_v, res=x2in, bn=512)
    x2 = _mm(mem_out, p['mpw'], p['mpb'], res=x1)

    rec = _rec_loss(fine_mem, p['cq'], p['cpw'], p['cpb'])

    h3 = _ln(x2, p['g3'], p['b3'])
    hr = _mm(h3, p['rw1'], p['rb1'], act="relu")
    rw2p = jnp.pad(p['rw2'], ((0, 0), (0, 128 - E)))
    rb2p = jnp.pad(p['rb2'], (0, 128 - E))
    scores_pad = _mm(hr, rw2p, rb2p, bn=128)
    scores = scores_pad[:, :E]
    ts, ti = jax.lax.top_k(scores.T, CAP)
    disp = _sc_gather(h3, ti.reshape(-1)).reshape(E, CAP, D)
    combined = _experts(disp, ti, ts, p['ew1'], p['eb1'], p['ew2'], p['eb2'])
    out = _add(x2, combined)

    imp = _imp_loss(scores_pad)
    aux = rec + imp
    return out.reshape(B, S, D), aux


# speedup vs baseline: 1.2567x; 1.0057x over previous
import functools
import math

import jax
import jax.numpy as jnp
import numpy as np
from jax.experimental import pallas as pl
from jax.experimental.pallas import tpu as pltpu
from jax.experimental.pallas import tpu_sc as plsc

B, S, D = 1, 2048, 1024
H = 16
HD = D // H
E = 8
DFF = 4 * D
CAP = math.ceil(1.25 * S / E)
MEM_LEN = 256
CMEM_LEN = 128
KF = 8
DFB = DFF // KF


def _ln(x, g, b):
    m = jnp.mean(x, axis=-1, keepdims=True)
    v = jnp.mean((x - m) ** 2, axis=-1, keepdims=True)
    return (x - m) / jnp.sqrt(v + 1e-5) * g + b


def _mm(a, w, bias=None, *, act=None, res=None, bm=256, bn=512):
    M, K = a.shape
    N = w.shape[1]
    bm = min(bm, M)
    bn = min(bn, N)
    operands = [a, w]
    specs = [
        pl.BlockSpec((bm, K), lambda i, j: (i, 0)),
        pl.BlockSpec((K, bn), lambda i, j: (0, j)),
    ]
    has_bias = bias is not None
    has_res = res is not None
    if has_bias:
        operands.append(bias.reshape(1, N))
        specs.append(pl.BlockSpec((1, bn), lambda i, j: (0, j)))
    if has_res:
        operands.append(res)
        specs.append(pl.BlockSpec((bm, bn), lambda i, j: (i, j)))

    def kfn(*refs):
        it = iter(refs)
        a_ref = next(it)
        w_ref = next(it)
        b_ref = next(it) if has_bias else None
        r_ref = next(it) if has_res else None
        o_ref = next(it)
        o = jnp.dot(a_ref[...], w_ref[...], preferred_element_type=jnp.float32)
        if has_bias:
            o = o + b_ref[...]
        if act == "relu":
            o = jnp.maximum(o, 0.0)
        if has_res:
            o = o + r_ref[...]
        o_ref[...] = o

    return pl.pallas_call(
        kfn,
        grid=(M // bm, N // bn),
        in_specs=specs,
        out_specs=pl.BlockSpec((bm, bn), lambda i, j: (i, j)),
        out_shape=jax.ShapeDtypeStruct((M, N), jnp.float32),
    )(*operands)


def _attn_scores_kernel(q_ref, k_ref, o_ref, *, bq):
    i = pl.program_id(1)
    row = i * bq + jax.lax.broadcasted_iota(jnp.int32, (bq, S), 0)
    col = jax.lax.broadcasted_iota(jnp.int32, (bq, S), 1)
    for ii in range(S // bq):
        @pl.when(i == ii)
        def _(ii=ii):
            w = (ii + 1) * bq
            s = jax.lax.dot_general(
                q_ref[0], k_ref[0, :w, :], (((1,), (1,)), ((), ())),
                preferred_element_type=jnp.float32,
            ) / math.sqrt(HD)
            if w < S:
                s = jnp.concatenate(
                    [s, jnp.full((bq, S - w), -1e30, jnp.float32)], axis=1)
            o_ref[0] = jnp.where(col > row, -1e30, s)


def _attn_scores(q, k, bq=512):
    return pl.pallas_call(
        functools.partial(_attn_scores_kernel, bq=bq),
        grid=(H, S // bq),
        in_specs=[
            pl.BlockSpec((1, bq, HD), lambda h, i: (h, i, 0)),
            pl.BlockSpec((1, S, HD), lambda h, i: (h, 0, 0)),
        ],
        out_specs=pl.BlockSpec((1, bq, S), lambda h, i: (h, i, 0)),
        out_shape=jax.ShapeDtypeStruct((H, S, S), jnp.float32),
    )(q, k)


def _attn_av_kernel(s_ref, l_ref, v_ref, o_ref):
    sv = s_ref[0]
    m = jnp.max(sv, axis=1, keepdims=True)
    e = jnp.exp(sv - m)
    w = e / jnp.transpose(l_ref[0, 0])
    o_ref[0] = jnp.dot(w, v_ref[0], preferred_element_type=jnp.float32)


def _attn_av(sc, l, v, bq=512):
    l4 = l.reshape(H, S // bq, 1, bq)
    blocks = []
    for i in range(S // bq):
        wd = (i + 1) * bq
        blk = pl.pallas_call(
            _attn_av_kernel,
            grid=(H,),
            in_specs=[
                pl.BlockSpec((1, bq, wd), lambda h, i=i: (h, i, 0)),
                pl.BlockSpec((1, 1, 1, bq), lambda h, i=i: (h, i, 0, 0)),
                pl.BlockSpec((1, wd, HD), lambda h: (h, 0, 0)),
            ],
            out_specs=pl.BlockSpec((1, bq, HD), lambda h: (h, 0, 0)),
            out_shape=jax.ShapeDtypeStruct((H, bq, HD), jnp.float32),
        )(sc, l4, v)
        blocks.append(blk)
    return jnp.concatenate(blocks, axis=1)


def _sc_gather(table, idx):
    nidx = idx.shape[0]
    info = plsc.get_sparse_core_info()
    nw = info.num_cores * info.num_subcores
    b_per_w = nidx // nw
    mesh = plsc.VectorSubcoreMesh(core_axis_name="c", subcore_axis_name="s")

    @functools.partial(
        pl.kernel, mesh=mesh,
        out_type=jax.ShapeDtypeStruct((nidx, D), jnp.float32),
        scratch_types=[
            pltpu.VMEM((b_per_w,), jnp.int32),
            pltpu.VMEM((b_per_w, D), jnp.float32),
            pltpu.SemaphoreType.DMA,
        ],
    )
    def k(table_hbm, idx_hbm, out_hbm, idx_v, rows_v, sem):
        wid = jax.lax.axis_index("s") * info.num_cores + jax.lax.axis_index("c")
        base = wid * b_per_w
        pltpu.sync_copy(idx_hbm.at[pl.ds(base, b_per_w)], idx_v)
        pltpu.async_copy(table_hbm.at[idx_v], rows_v, sem).wait()
        pltpu.sync_copy(rows_v, out_hbm.at[pl.ds(base, b_per_w)])

    return k(table, idx)


def _expert_kernel(disp_ref, ti_ref, ts_ref, ew1_ref, eb1_ref,
                   ew2_ref, eb2_ref, o_ref, acc_ref):
    e = pl.program_id(0)
    kf = pl.program_id(1)

    @pl.when(jnp.logical_and(e == 0, kf == 0))
    def _():
        o_ref[...] = jnp.zeros_like(o_ref)

    rows = jax.lax.broadcasted_iota(jnp.int32, (S, CAP), 0)
    oh = (rows == ti_ref[0]).astype(jnp.float32)

    h1 = jnp.maximum(
        jnp.dot(disp_ref[0], ew1_ref[0], preferred_element_type=jnp.float32)
        + eb1_ref[0, 0], 0.0)
    contrib = jnp.dot(h1, ew2_ref[0], preferred_element_type=jnp.float32)

    @pl.when(kf == 0)
    def _():
        acc_ref[...] = contrib

    @pl.when(kf > 0)
    def _():
        acc_ref[...] = acc_ref[...] + contrib

    @pl.when(kf == KF - 1)
    def _():
        eo = acc_ref[...] + eb2_ref[0]
        o_ref[...] = o_ref[...] + jnp.dot(
            oh * ts_ref[0], eo, preferred_element_type=jnp.float32)


def _experts(disp, ti, ts, ew1, eb1, ew2, eb2):
    return pl.pallas_call(
        _expert_kernel,
        grid=(E, KF),
        in_specs=[
            pl.BlockSpec((1, CAP, D), lambda e, kf: (e, 0, 0)),
            pl.BlockSpec((1, 1, CAP), lambda e, kf: (e, 0, 0)),
            pl.BlockSpec((1, 1, CAP), lambda e, kf: (e, 0, 0)),
            pl.BlockSpec((1, D, DFB), lambda e, kf: (e, 0, kf)),
            pl.BlockSpec((1, 1, 1, DFB), lambda e, kf: (e, kf, 0, 0)),
            pl.BlockSpec((1, DFB, D), lambda e, kf: (e, kf, 0)),
            pl.BlockSpec((1, 1, D), lambda e, kf: (e, 0, 0)),
        ],
        out_specs=pl.BlockSpec((S, D), lambda e, kf: (0, 0)),
        out_shape=jax.ShapeDtypeStruct((S, D), jnp.float32),
        scratch_shapes=[
            pltpu.VMEM((CAP, D), jnp.float32),
        ],
    )(
        disp,
        ti.reshape(E, 1, CAP),
        ts.reshape(E, 1, CAP),
        ew1,
        eb1.reshape(E, KF, 1, DFB),
        ew2,
        eb2.reshape(E, 1, D),
    )


def _add_kernel(a_ref, b_ref, o_ref):
    o_ref[...] = a_ref[...] + b_ref[...]


def _add(a, b, bm=256):
    M, N = a.shape
    return pl.pallas_call(
        _add_kernel,
        grid=(M // bm,),
        in_specs=[pl.BlockSpec((bm, N), lambda i: (i, 0))] * 2,
        out_specs=pl.BlockSpec((bm, N), lambda i: (i, 0)),
        out_shape=jax.ShapeDtypeStruct((M, N), jnp.float32),
    )(a, b)


def _rec_kernel(ev_ref, od_ref, cq_ref, cpw_ref, cpb_ref, it_ref, fm_ref, o_ref):
    inv = 1.0 / math.sqrt(D)
    ev = ev_ref[...]
    od = od_ref[...]
    cq = cq_ref[...]
    se = jnp.sum(ev * cq, axis=1, keepdims=True) * inv
    so = jnp.sum(od * cq, axis=1, keepdims=True) * inv
    m = jnp.maximum(se, so)
    ae = jnp.exp(se - m)
    ao = jnp.exp(so - m)
    mix = (ae * ev + ao * od) / (ae + ao)
    comp = jnp.dot(mix, cpw_ref[...], preferred_element_type=jnp.float32) + cpb_ref[...]
    dec = jnp.dot(it_ref[...], comp, preferred_element_type=jnp.float32)
    d = dec - fm_ref[...]
    ssq = jnp.sum(jnp.sum(d * d, axis=1, keepdims=True), axis=0, keepdims=True)
    o_ref[...] = ssq * (1.0 / (MEM_LEN * D))


def _interp_mat():
    L, out_len = MEM_LEN // 2, MEM_LEN
    pos = (np.arange(out_len, dtype=np.float64) + 0.5) * L / out_len - 0.5
    pos = np.clip(pos, 0.0, L - 1.0)
    lo = np.floor(pos).astype(np.int32)
    hi = np.clip(lo + 1, 0, L - 1)
    w = (pos - lo).astype(np.float32)
    mat = np.zeros((out_len, L), np.float32)
    mat[np.arange(out_len), lo] += 1.0 - w
    mat[np.arange(out_len), hi] += w
    return jnp.asarray(mat)


def _rec_loss(fine_mem, cq, cpw, cpb):
    fm3 = fine_mem.reshape(MEM_LEN // 2, 2, D)
    return pl.pallas_call(
        _rec_kernel,
        out_shape=jax.ShapeDtypeStruct((1, 1), jnp.float32),
    )(fm3[:, 0, :], fm3[:, 1, :], cq.reshape(1, D), cpw, cpb.reshape(1, D),
      _interp_mat(), fine_mem)[0, 0]


def _imp_kernel(s_ref, o_ref):
    x = s_ref[...]
    colid = jax.lax.broadcasted_iota(jnp.int32, x.shape, 1)
    valid = colid < E
    xm = jnp.where(valid, x, -1e30)
    m = jnp.max(xm, axis=1, keepdims=True)
    ex = jnp.where(valid, jnp.exp(x - m), 0.0)
    p = ex / jnp.sum(ex, axis=1, keepdims=True)
    imp = jnp.sum(p, axis=0, keepdims=True)
    mean = jnp.sum(imp, axis=1, keepdims=True) / E
    dv = jnp.where(valid[:1, :], imp - mean, 0.0)
    var = jnp.sum(dv * dv, axis=1, keepdims=True) / (E - 1)
    o_ref[...] = var / (mean * mean + 1e-6)


def _imp_loss(scores_pad):
    return pl.pallas_call(
        _imp_kernel,
        out_shape=jax.ShapeDtypeStruct((1, 1), jnp.float32),
    )(scores_pad)[0, 0]


def kernel(x, fine_mem, cmem, params):
    p = params
    xf = x.reshape(S, D)

    h = _ln(xf, p['g1'], p['b1'])
    wqkv = jnp.concatenate([p['Wq'], p['Wk'], p['Wv']], axis=1)
    qkv = _mm(h, wqkv)
    q = qkv[:, :D].reshape(S, H, HD).transpose(1, 0, 2)
    k = qkv[:, D:2 * D].reshape(S, H, HD).transpose(1, 0, 2)
    v = qkv[:, 2 * D:].reshape(S, H, HD).transpose(1, 0, 2)
    sc = _attn_scores(q, k)
    lsum = jnp.sum(jnp.exp(sc - jnp.max(sc, axis=-1, keepdims=True)), axis=-1)
    ao = _attn_av(sc, lsum, v).transpose(1, 0, 2).reshape(S, D)
    x1 = _mm(ao, p['Wo'], res=xf)

    fkv = _mm(fine_mem, jnp.concatenate([p['kpw'], p['vpw']], axis=1),
              jnp.concatenate([p['kpb'], p['vpb']]))
    ckv = _mm(cmem, jnp.concatenate([p['cmkw'], p['cmvw']], axis=1),
              jnp.concatenate([p['cmkb'], p['cmvb']]), bm=128)
    mem_k = jnp.concatenate([fkv[:, :D], ckv[:, :D]], axis=0)
    mem_v = jnp.concatenate([fkv[:, D:], ckv[:, D:]], axis=0)
    x2in = _ln(x1, p['g2'], p['b2'])
    qp = _mm(x2in, p['ckw'], p['ckb'])
    ms = _mm(qp, mem_k.T) / math.sqrt(D)
    ma = jax.nn.softmax(ms, axis=-1)
    mem_out = _mm(ma, mem_v, res=x2in, bn=512)
    x2 = _mm(mem_out, p['mpw'], p['mpb'], res=x1)

    rec = _rec_loss(fine_mem, p['cq'], p['cpw'], p['cpb'])

    h3 = _ln(x2, p['g3'], p['b3'])
    hr = _mm(h3, p['rw1'], p['rb1'], act="relu")
    rw2p = jnp.pad(p['rw2'], ((0, 0), (0, 128 - E)))
    rb2p = jnp.pad(p['rb2'], (0, 128 - E))
    scores_pad = _mm(hr, rw2p, rb2p, bn=128)
    scores = scores_pad[:, :E]
    ts, ti = jax.lax.top_k(scores.T, CAP)
    disp = _sc_gather(h3, ti.reshape(-1)).reshape(E, CAP, D)
    combined = _experts(disp, ti, ts, p['ew1'], p['eb1'], p['ew2'], p['eb2'])
    out = _add(x2, combined)

    imp = _imp_loss(scores_pad)
    aux = rec + imp
    return out.reshape(B, S, D), aux
